# baseline (speedup 1.0000x reference)
import functools

import jax
import jax.numpy as jnp
from jax import lax
from jax.experimental import pallas as pl
from jax.experimental.pallas import tpu as pltpu
from jax.experimental.pallas import tpu_sc as plsc

B = 16
T = 50
P = 8
H = 64
W = 64
N = B * T
SLICE = H * W * P
NC = 2
NS = 16
NWORK = NC * NS
RPW = N // NWORK


def _sc_body(xd_h, yd_h, dx_h, dy_h, ox_h, oy_h, z_h, out_h,
             xv, yv, dxv, dyv, oxv, oyv, buf0, buf1, sem0, sem1):
    w = lax.axis_index("s") * NC + lax.axis_index("c")
    base = w * RPW

    pltpu.sync_copy(xd_h.at[pl.ds(base * 16, RPW * 16)], xv)
    pltpu.sync_copy(yd_h.at[pl.ds(base * 16, RPW * 16)], yv)
    pltpu.sync_copy(dx_h.at[pl.ds(base * 16, RPW * 16)], dxv)
    pltpu.sync_copy(dy_h.at[pl.ds(base * 16, RPW * 16)], dyv)
    pltpu.sync_copy(ox_h.at[pl.ds(base * 16, RPW * 16)], oxv)
    pltpu.sync_copy(oy_h.at[pl.ds(base * 16, RPW * 16)], oyv)
    pltpu.sync_copy(z_h, buf0)
    pltpu.sync_copy(z_h, buf1)

    lane = lax.iota(jnp.int32, 16)
    lane_p = lane & 7
    mask_lo = lane < 8
    ones = jnp.full((16,), 1.0, jnp.float32)
    zeros_v = jnp.zeros((16,), jnp.float32)

    bufs = (buf0, buf1)
    sems = (sem0, sem1)
    prev = [None, None]
    handles = [None] * RPW
    for s in range(RPW):
        b = s & 1
        buf = bufs[b]
        if s >= 2:
            handles[s - 2].wait()
            addr_old, msk_old = prev[b]
            plsc.store_scatter(buf, [addr_old], zeros_v, mask=msk_old)
        sl = pl.ds(s * 16, 16)
        cf = xv[sl] / dxv[sl] + oxv[sl]
        rf = yv[sl] / dyv[sl] + oyv[sl]
        ci = cf.astype(jnp.int32)
        ri = rf.astype(jnp.int32)
        ok = mask_lo & (ci >= 0) & (ci < W) & (ri >= 0) & (ri < H)
        ciq = jnp.clip(ci, 0, W - 1)
        riq = jnp.clip(ri, 0, H - 1)
        addr = riq * (W * P) + ciq * P + lane_p
        plsc.store_scatter(buf, [addr], ones, mask=ok)
        handles[s] = pltpu.async_copy(
            buf, out_h.at[pl.ds((base + s) * SLICE, SLICE)], sems[b])
        prev[b] = (addr, ok)
    handles[RPW - 2].wait()
    handles[RPW - 1].wait()


_sc_fn = functools.partial(
    pl.kernel,
    out_type=jax.ShapeDtypeStruct((N * SLICE,), jnp.float32),
    mesh=plsc.VectorSubcoreMesh(core_axis_name="c", subcore_axis_name="s"),
    compiler_params=pltpu.CompilerParams(needs_layout_passes=False),
    scratch_types=[
        pltpu.VMEM((RPW * 16,), jnp.float32),
        pltpu.VMEM((RPW * 16,), jnp.float32),
        pltpu.VMEM((RPW * 16,), jnp.float32),
        pltpu.VMEM((RPW * 16,), jnp.float32),
        pltpu.VMEM((RPW * 16,), jnp.float32),
        pltpu.VMEM((RPW * 16,), jnp.float32),
        pltpu.VMEM((SLICE,), jnp.float32),
        pltpu.VMEM((SLICE,), jnp.float32),
        pltpu.SemaphoreType.DMA,
        pltpu.SemaphoreType.DMA,
    ],
)(_sc_body)


def kernel(x, resolution, origin):
    pts = x.reshape(N, P, 2)
    xd = jnp.tile(pts[:, :, 0], (1, 2)).reshape(-1)
    yd = jnp.tile(pts[:, :, 1], (1, 2)).reshape(-1)
    res = resolution.reshape(N, 2)
    org = origin.reshape(N, 2)
    dx = jnp.tile(res[:, 0:1], (1, 16)).reshape(-1)
    dy = jnp.tile(res[:, 1:2], (1, 16)).reshape(-1)
    ox = jnp.tile(org[:, 1:2], (1, 16)).reshape(-1)
    oy = jnp.tile(org[:, 0:1], (1, 16)).reshape(-1)
    z = jnp.zeros((SLICE,), jnp.float32)

    out = _sc_fn(xd, yd, dx, dy, ox, oy, z)
    return out.reshape(B, T, H, W, P)


# speedup vs baseline: 17.4197x; 17.4197x over previous
import jax
import jax.numpy as jnp
from jax.experimental import pallas as pl

B = 16
T = 50
P = 8
H = 64
W = 64
N = B * T
G = 40


def _body(xr, yr, dxr, dyr, oxr, oyr, out_ref):
    coli = (xr[...] / dxr[...] + oxr[...]).astype(jnp.int32)
    rowi = (yr[...] / dyr[...] + oyr[...]).astype(jnp.int32)
    inb = (coli >= 0) & (coli < W) & (rowi >= 0) & (rowi < H)
    tgt_r = jnp.where(inb, rowi, -1)
    hio = jax.lax.broadcasted_iota(jnp.int32, (G, H, P, W), 1)
    wio = jax.lax.broadcasted_iota(jnp.int32, (G, H, P, W), 3)
    hit = (hio == tgt_r[:, None, :, None]) & (wio == coli[:, None, :, None])
    out_ref[...] = hit.astype(jnp.float32)


def kernel(x, resolution, origin):
    pts = x.reshape(N, P, 2)
    xc = pts[:, :, 0]
    yc = pts[:, :, 1]
    res = resolution.reshape(N, 2)
    org = origin.reshape(N, 2)
    dx = jnp.tile(res[:, 0:1], (1, P))
    dy = jnp.tile(res[:, 1:2], (1, P))
    ox = jnp.tile(org[:, 1:2], (1, P))
    oy = jnp.tile(org[:, 0:1], (1, P))

    out = pl.pallas_call(
        _body,
        grid=(N // G,),
        in_specs=[pl.BlockSpec((G, P), lambda i: (i, 0))] * 6,
        out_specs=pl.BlockSpec((G, H, P, W), lambda i: (i, 0, 0, 0)),
        out_shape=jax.ShapeDtypeStruct((N, H, P, W), jnp.float32),
    )(xc, yc, dx, dy, ox, oy)
    out5 = out.reshape(B, T, H, P, W)
    return jnp.transpose(out5, (0, 1, 2, 4, 3))
